# baseline (speedup 1.0000x reference)
import jax
import jax.numpy as jnp
import numpy as np
from jax.experimental import pallas as pl

BLOCK = 1024
A_STRIDE = 64


def _recon_kernel(sa_ref, ca_ref, sb_ref, cb_ref, out_ref):
    sbv = sb_ref[...]
    cbv = cb_ref[...]
    parts = []
    for aa in range(BLOCK // A_STRIDE):
        row_s = sa_ref[aa, :][None, :]
        row_c = ca_ref[aa, :][None, :]
        parts.append(row_s * cbv + row_c * sbv)
    tab = jnp.concatenate(parts, axis=0)
    out_ref[...] = jnp.broadcast_to(tab[None, :, :], out_ref.shape)


def kernel(inputs, pos_table):
    batch, n_seq = inputs.shape
    d_model = pos_table.shape[1]
    n_a = n_seq // A_STRIDE
    a_per_block = BLOCK // A_STRIDE

    col = np.arange(d_model)
    w = np.power(10000.0, -2.0 * (col // 2) / d_model)
    phase = (col % 2) * (np.pi / 2.0)
    a_ang = np.outer(np.arange(n_a) * A_STRIDE, w) + phase
    b_ang = np.outer(np.arange(A_STRIDE), w)
    sa = jnp.asarray(np.sin(a_ang), dtype=jnp.float32)
    ca = jnp.asarray(np.cos(a_ang), dtype=jnp.float32)
    sb = jnp.asarray(np.sin(b_ang), dtype=jnp.float32)
    cb = jnp.asarray(np.cos(b_ang), dtype=jnp.float32)

    grid = (n_seq // BLOCK,)
    return pl.pallas_call(
        _recon_kernel,
        grid=grid,
        in_specs=[
            pl.BlockSpec((a_per_block, d_model), lambda i: (i, 0)),
            pl.BlockSpec((a_per_block, d_model), lambda i: (i, 0)),
            pl.BlockSpec((A_STRIDE, d_model), lambda i: (0, 0)),
            pl.BlockSpec((A_STRIDE, d_model), lambda i: (0, 0)),
        ],
        out_specs=pl.BlockSpec((batch, BLOCK, d_model), lambda i: (0, i, 0)),
        out_shape=jax.ShapeDtypeStruct((batch, n_seq, d_model), pos_table.dtype),
    )(sa, ca, sb, cb)


# speedup vs baseline: 1.0553x; 1.0553x over previous
import jax
import jax.numpy as jnp
import numpy as np
from jax.experimental import pallas as pl
from jax.experimental.pallas import tpu as pltpu

BLOCK = 512
A_STRIDE = 64
NSLOTS = 2


def _recon_dma_kernel(sa_ref, ca_ref, sb_ref, cb_ref, out_hbm, scratch, sems):
    i = pl.program_id(0)
    nsteps = pl.num_programs(0)
    batch = out_hbm.shape[0]
    slot = jax.lax.rem(i, NSLOTS)

    def _copies(step, s):
        return [
            pltpu.make_async_copy(
                scratch.at[pl.ds(s * BLOCK, BLOCK), :],
                out_hbm.at[b, pl.ds(step * BLOCK, BLOCK), :],
                sems.at[s, b],
            )
            for b in range(batch)
        ]

    @pl.when(i >= NSLOTS)
    def _wait_prev():
        for c in _copies(i - NSLOTS, slot):
            c.wait()

    sbv = sb_ref[...]
    cbv = cb_ref[...]
    parts = []
    for aa in range(BLOCK // A_STRIDE):
        row_s = sa_ref[aa, :][None, :]
        row_c = ca_ref[aa, :][None, :]
        parts.append(row_s * cbv + row_c * sbv)
    scratch[pl.ds(slot * BLOCK, BLOCK), :] = jnp.concatenate(parts, axis=0)

    for c in _copies(i, slot):
        c.start()

    @pl.when(i == nsteps - 1)
    def _drain():
        for s_off in range(1, NSLOTS + 1):
            step = i - NSLOTS + s_off
            s = jax.lax.rem(jnp.int32(step), NSLOTS)
            for c in _copies(step, s):
                c.wait()


def kernel(inputs, pos_table):
    batch, n_seq = inputs.shape
    d_model = pos_table.shape[1]
    n_a = n_seq // A_STRIDE
    a_per_block = BLOCK // A_STRIDE

    col = np.arange(d_model)
    w = np.power(10000.0, -2.0 * (col // 2) / d_model)
    phase = (col % 2) * (np.pi / 2.0)
    a_ang = np.outer(np.arange(n_a) * A_STRIDE, w) + phase
    b_ang = np.outer(np.arange(A_STRIDE), w)
    sa = jnp.asarray(np.sin(a_ang), dtype=jnp.float32)
    ca = jnp.asarray(np.cos(a_ang), dtype=jnp.float32)
    sb = jnp.asarray(np.sin(b_ang), dtype=jnp.float32)
    cb = jnp.asarray(np.cos(b_ang), dtype=jnp.float32)

    grid = (n_seq // BLOCK,)
    return pl.pallas_call(
        _recon_dma_kernel,
        grid=grid,
        in_specs=[
            pl.BlockSpec((a_per_block, d_model), lambda i: (i, 0)),
            pl.BlockSpec((a_per_block, d_model), lambda i: (i, 0)),
            pl.BlockSpec((A_STRIDE, d_model), lambda i: (0, 0)),
            pl.BlockSpec((A_STRIDE, d_model), lambda i: (0, 0)),
        ],
        out_specs=pl.BlockSpec(memory_space=pl.ANY),
        out_shape=jax.ShapeDtypeStruct((batch, n_seq, d_model), pos_table.dtype),
        scratch_shapes=[
            pltpu.VMEM((NSLOTS * BLOCK, d_model), jnp.float32),
            pltpu.SemaphoreType.DMA((NSLOTS, 4)),
        ],
    )(sa, ca, sb, cb)


# speedup vs baseline: 1.1470x; 1.0869x over previous
import jax
import jax.numpy as jnp
import numpy as np
from jax.experimental import pallas as pl
from jax.experimental.pallas import tpu as pltpu

BLOCK = 256
A_STRIDE = 32
NSLOTS = 2


def _recon_dma_kernel(sa_ref, ca_ref, sb_ref, cb_ref, out_hbm, scratch, sems):
    i = pl.program_id(0)
    nsteps = pl.num_programs(0)
    batch = out_hbm.shape[0]
    slot = jax.lax.rem(i, NSLOTS)

    def _copies(step, s):
        return [
            pltpu.make_async_copy(
                scratch.at[pl.ds(s * BLOCK, BLOCK), :],
                out_hbm.at[b, pl.ds(step * BLOCK, BLOCK), :],
                sems.at[s, b],
            )
            for b in range(batch)
        ]

    @pl.when(i >= NSLOTS)
    def _wait_prev():
        for c in _copies(i - NSLOTS, slot):
            c.wait()

    sbv = sb_ref[...]
    cbv = cb_ref[...]
    parts = []
    for aa in range(BLOCK // A_STRIDE):
        row_s = sa_ref[aa, :][None, :]
        row_c = ca_ref[aa, :][None, :]
        parts.append(row_s * cbv + row_c * sbv)
    scratch[pl.ds(slot * BLOCK, BLOCK), :] = jnp.concatenate(parts, axis=0)

    for c in _copies(i, slot):
        c.start()

    @pl.when(i == nsteps - 1)
    def _drain():
        for s_off in range(1, NSLOTS + 1):
            step = i - NSLOTS + s_off
            s = jax.lax.rem(jnp.int32(step), NSLOTS)
            for c in _copies(step, s):
                c.wait()


def kernel(inputs, pos_table):
    batch, n_seq = inputs.shape
    d_model = pos_table.shape[1]
    n_a = n_seq // A_STRIDE
    a_per_block = BLOCK // A_STRIDE

    col = np.arange(d_model)
    w = np.power(10000.0, -2.0 * (col // 2) / d_model)
    phase = (col % 2) * (np.pi / 2.0)
    a_ang = np.outer(np.arange(n_a) * A_STRIDE, w) + phase
    b_ang = np.outer(np.arange(A_STRIDE), w)
    sa = jnp.asarray(np.sin(a_ang), dtype=jnp.float32)
    ca = jnp.asarray(np.cos(a_ang), dtype=jnp.float32)
    sb = jnp.asarray(np.sin(b_ang), dtype=jnp.float32)
    cb = jnp.asarray(np.cos(b_ang), dtype=jnp.float32)

    grid = (n_seq // BLOCK,)
    return pl.pallas_call(
        _recon_dma_kernel,
        grid=grid,
        in_specs=[
            pl.BlockSpec((a_per_block, d_model), lambda i: (i, 0)),
            pl.BlockSpec((a_per_block, d_model), lambda i: (i, 0)),
            pl.BlockSpec((A_STRIDE, d_model), lambda i: (0, 0)),
            pl.BlockSpec((A_STRIDE, d_model), lambda i: (0, 0)),
        ],
        out_specs=pl.BlockSpec(memory_space=pl.ANY),
        out_shape=jax.ShapeDtypeStruct((batch, n_seq, d_model), pos_table.dtype),
        scratch_shapes=[
            pltpu.VMEM((NSLOTS * BLOCK, d_model), jnp.float32),
            pltpu.SemaphoreType.DMA((NSLOTS, 4)),
        ],
    )(sa, ca, sb, cb)
